# Initial kernel scaffold; baseline (speedup 1.0000x reference)
import jax
import jax.numpy as jnp
from jax.experimental import pallas as pl


def kernel(x, edge_index, Wl1, bl1, Wr1, Wl2, bl2, Wr2):
    raise NotImplementedError("write your pallas kernel here")


# speedup vs baseline: 2.0205x; 2.0205x over previous
import jax
import jax.numpy as jnp
from jax import lax
from jax.experimental import pallas as pl
from jax.experimental.pallas import tpu as pltpu
from jax.experimental.pallas import tpu_sc as plsc

N = 10000
E = 160000
D_IN = 256
D_H = 512
D_OUT = 256

NC = 2
NS = 16
NW = NC * NS
CHUNK = 128
CH_PER_TILE = 40
EDGES_PER_TILE = CHUNK * CH_PER_TILE
E_PAD = EDGES_PER_TILE * NW
N_PAD = 10240
ROWS_PER_TILE = N_PAD // NS
TRASH = N_PAD - 8
HALF = 128

def _build_sc_segsum(mesh, n_pass):
  def body(*refs):
    xs = refs[:n_pass]
    (src2, dst2, zrows, p_out, src_v, dst_v, rows_v, acc_sh, sem) = refs[n_pass:]
    c = lax.axis_index("c")
    s = lax.axis_index("s")
    w = c * NS + s
    pltpu.sync_copy(src2.at[pl.ds(w * CH_PER_TILE, CH_PER_TILE)], src_v)
    pltpu.sync_copy(dst2.at[pl.ds(w * CH_PER_TILE, CH_PER_TILE)], dst_v)

    for p in range(n_pass):
      xh = xs[p]
      pltpu.sync_copy(zrows, acc_sh.at[pl.ds(s * ROWS_PER_TILE, ROWS_PER_TILE)])
      plsc.subcore_barrier()

      @pl.loop(0, CH_PER_TILE)
      def chunk_body(j):
        pltpu.async_copy(xh.at[src_v.at[j]], rows_v, sem).wait()
        pltpu.sync_copy(rows_v, acc_sh.at[dst_v.at[j]], add=True)

      plsc.subcore_barrier()
      rows = pl.ds(s * ROWS_PER_TILE, ROWS_PER_TILE)
      pltpu.sync_copy(acc_sh.at[rows], p_out.at[c, p, rows])

  return pl.kernel(
      body,
      out_type=jax.ShapeDtypeStruct((NC, n_pass, N_PAD, HALF), jnp.float32),
      mesh=mesh,
      scratch_types=[
          pltpu.VMEM((CH_PER_TILE, CHUNK), jnp.int32),
          pltpu.VMEM((CH_PER_TILE, CHUNK), jnp.int32),
          pltpu.VMEM((CHUNK, HALF), jnp.float32),
          pltpu.VMEM_SHARED((N_PAD, HALF), jnp.float32),
          pltpu.SemaphoreType.DMA,
      ],
      name="sc_segsum%d" % n_pass,
  )


_sc_built = {}


def _sc_kernels():
  if not _sc_built:
    mesh = plsc.VectorSubcoreMesh(core_axis_name="c", subcore_axis_name="s",
                                  num_cores=NC, num_subcores=NS)
    _sc_built["segsum"] = _build_sc_segsum(mesh, 2)
    _sc_built["deg"] = _build_sc_segsum(mesh, 1)
  return _sc_built["segsum"], _sc_built["deg"]

_ROWS = 1024
_GRID = N_PAD // _ROWS


def _tc_layer1_body(p_ref, deg_ref, x_ref, wl1_ref, bl1_ref, wr1_ref,
                    wl2_ref, wr2_ref, mlo_ref, mhi_ref, hr2_ref):
  p = p_ref[...]
  agg = jnp.concatenate([p[0, 0] + p[1, 0], p[0, 1] + p[1, 1]], axis=-1)
  dg = deg_ref[0] + deg_ref[1]
  aggn = agg / jnp.maximum(dg, 1.0)
  h = jnp.maximum(
      jnp.dot(aggn, wl1_ref[...], preferred_element_type=jnp.float32)
      + bl1_ref[...]
      + jnp.dot(x_ref[...], wr1_ref[...], preferred_element_type=jnp.float32),
      0.0)
  m = jnp.dot(h, wl2_ref[...], preferred_element_type=jnp.float32)
  mlo_ref[...] = m[:, :HALF]
  mhi_ref[...] = m[:, HALF:]
  hr2_ref[...] = jnp.dot(h, wr2_ref[...], preferred_element_type=jnp.float32)


def _tc_layer1(P1, deg2, x_pad, Wl1T, bl1r, Wr1T, Wl2T, Wr2T):
  return pl.pallas_call(
      _tc_layer1_body,
      grid=(_GRID,),
      in_specs=[
          pl.BlockSpec((NC, 2, _ROWS, HALF), lambda i: (0, 0, i, 0)),
          pl.BlockSpec((NC, _ROWS, 1), lambda i: (0, i, 0)),
          pl.BlockSpec((_ROWS, D_IN), lambda i: (i, 0)),
          pl.BlockSpec((D_IN, D_H), lambda i: (0, 0)),
          pl.BlockSpec((1, D_H), lambda i: (0, 0)),
          pl.BlockSpec((D_IN, D_H), lambda i: (0, 0)),
          pl.BlockSpec((D_H, D_OUT), lambda i: (0, 0)),
          pl.BlockSpec((D_H, D_OUT), lambda i: (0, 0)),
      ],
      out_specs=[
          pl.BlockSpec((_ROWS, HALF), lambda i: (i, 0)),
          pl.BlockSpec((_ROWS, HALF), lambda i: (i, 0)),
          pl.BlockSpec((_ROWS, D_OUT), lambda i: (i, 0)),
      ],
      out_shape=[
          jax.ShapeDtypeStruct((N_PAD, HALF), jnp.float32),
          jax.ShapeDtypeStruct((N_PAD, HALF), jnp.float32),
          jax.ShapeDtypeStruct((N_PAD, D_OUT), jnp.float32),
      ],
      name="tc_layer1",
  )(P1, deg2, x_pad, Wl1T, bl1r, Wr1T, Wl2T, Wr2T)


def _tc_layer2_body(p_ref, deg_ref, hr2_ref, bl2_ref, out_ref):
  p = p_ref[...]
  agg = jnp.concatenate([p[0, 0] + p[1, 0], p[0, 1] + p[1, 1]], axis=-1)
  dg = deg_ref[0] + deg_ref[1]
  out_ref[...] = agg / jnp.maximum(dg, 1.0) + bl2_ref[...] + hr2_ref[...]


def _tc_layer2(P2, deg2, hr2, bl2r):
  return pl.pallas_call(
      _tc_layer2_body,
      grid=(_GRID,),
      in_specs=[
          pl.BlockSpec((NC, 2, _ROWS, HALF), lambda i: (0, 0, i, 0)),
          pl.BlockSpec((NC, _ROWS, 1), lambda i: (0, i, 0)),
          pl.BlockSpec((_ROWS, D_OUT), lambda i: (i, 0)),
          pl.BlockSpec((1, D_OUT), lambda i: (0, 0)),
      ],
      out_specs=pl.BlockSpec((_ROWS, D_OUT), lambda i: (i, 0)),
      out_shape=jax.ShapeDtypeStruct((N_PAD, D_OUT), jnp.float32),
      name="tc_layer2",
  )(P2, deg2, hr2, bl2r)


def kernel(x, edge_index, Wl1, bl1, Wr1, Wl2, bl2, Wr2):
  src = edge_index[0]
  dst = edge_index[1]
  pad = E_PAD - E
  src2 = jnp.concatenate([src, jnp.zeros((pad,), jnp.int32)]).reshape(
      E_PAD // CHUNK, CHUNK)
  dst2 = jnp.concatenate([dst, jnp.full((pad,), TRASH, jnp.int32)]).reshape(
      E_PAD // CHUNK, CHUNK)
  x_lo = x[:, :HALF]
  x_hi = x[:, HALF:]
  x_pad = jnp.concatenate(
      [x, jnp.zeros((N_PAD - N, D_IN), x.dtype)], axis=0)
  zh = jnp.zeros((ROWS_PER_TILE, HALF), jnp.float32)
  probe = jnp.ones((N, HALF), jnp.float32)

  sc_segsum, sc_deg = _sc_kernels()
  degP = sc_deg(probe, src2, dst2, zh)
  P1 = sc_segsum(x_lo, x_hi, src2, dst2, zh)
  deg2 = degP[:, 0, :, 0:1]
  m_lo, m_hi, hr2 = _tc_layer1(P1, deg2, x_pad, Wl1.T, bl1.reshape(1, -1),
                               Wr1.T, Wl2.T, Wr2.T)
  P2 = sc_segsum(m_lo, m_hi, src2, dst2, zh)
  out_pad = _tc_layer2(P2, deg2, hr2, bl2.reshape(1, -1))
  return out_pad[:N]


# speedup vs baseline: 2.6353x; 1.3043x over previous
import jax
import jax.numpy as jnp
from jax import lax
from jax.experimental import pallas as pl
from jax.experimental.pallas import tpu as pltpu
from jax.experimental.pallas import tpu_sc as plsc

N = 10000
E = 160000
D_IN = 256
D_H = 512
D_OUT = 256

NC = 2
NS = 16
NW = NC * NS
CHUNK = 128
CH_PER_TILE = 40
EDGES_PER_TILE = CHUNK * CH_PER_TILE
E_PAD = EDGES_PER_TILE * NW
N_PAD = 10240
ROWS_PER_TILE = N_PAD // NS
TRASH = N_PAD - 8
HALF = 128

def _build_sc_segsum(mesh, n_pass, deg_pass):
  n_slab = n_pass + (1 if deg_pass else 0)

  def body(*refs):
    xs = refs[:n_pass]
    (src2, dst2, zrows, ones_h, p_out,
     src_v, dst_v, rows_a, rows_b, acc_sh,
     sem_ga, sem_gb, sem_sa, sem_sb) = refs[n_pass:]
    c = lax.axis_index("c")
    s = lax.axis_index("s")
    w = c * NS + s
    pltpu.sync_copy(src2.at[pl.ds(w * CH_PER_TILE, CH_PER_TILE)], src_v)
    pltpu.sync_copy(dst2.at[pl.ds(w * CH_PER_TILE, CH_PER_TILE)], dst_v)

    def zero_acc():
      pltpu.sync_copy(zrows, acc_sh.at[pl.ds(s * ROWS_PER_TILE, ROWS_PER_TILE)])

    def flush(slab):
      rows = pl.ds(s * ROWS_PER_TILE, ROWS_PER_TILE)
      pltpu.sync_copy(acc_sh.at[rows], p_out.at[c, slab, rows])

    for p in range(n_pass):
      xh = xs[p]
      zero_acc()
      plsc.subcore_barrier()

      pltpu.async_copy(xh.at[src_v.at[0]], rows_a, sem_ga)
      pltpu.async_copy(xh.at[src_v.at[1]], rows_b, sem_gb)

      @pl.loop(0, CH_PER_TILE, step=2)
      def chunk2(j):
        pltpu.make_async_copy(xh.at[src_v.at[j]], rows_a, sem_ga).wait()
        pltpu.async_copy(rows_a, acc_sh.at[dst_v.at[j]], sem_sa, add=True)
        pltpu.make_async_copy(xh.at[src_v.at[j + 1]], rows_b, sem_gb).wait()
        pltpu.async_copy(rows_b, acc_sh.at[dst_v.at[j + 1]], sem_sb, add=True)
        pltpu.make_async_copy(rows_a, acc_sh.at[dst_v.at[j]], sem_sa).wait()

        @pl.when(j + 2 < CH_PER_TILE)
        def _():
          pltpu.async_copy(xh.at[src_v.at[j + 2]], rows_a, sem_ga)

        pltpu.make_async_copy(rows_b, acc_sh.at[dst_v.at[j + 1]], sem_sb).wait()

        @pl.when(j + 3 < CH_PER_TILE)
        def _():
          pltpu.async_copy(xh.at[src_v.at[j + 3]], rows_b, sem_gb)

      plsc.subcore_barrier()
      flush(p)

    if deg_pass:
      pltpu.sync_copy(ones_h, rows_a)
      pltpu.sync_copy(ones_h, rows_b)
      zero_acc()
      plsc.subcore_barrier()

      @pl.loop(0, CH_PER_TILE, step=2)
      def dchunk(j):
        pltpu.async_copy(rows_a, acc_sh.at[dst_v.at[j]], sem_sa, add=True)
        pltpu.async_copy(rows_b, acc_sh.at[dst_v.at[j + 1]], sem_sb, add=True)
        pltpu.make_async_copy(rows_a, acc_sh.at[dst_v.at[j]], sem_sa).wait()
        pltpu.make_async_copy(rows_b, acc_sh.at[dst_v.at[j + 1]], sem_sb).wait()

      plsc.subcore_barrier()
      flush(n_pass)

  return pl.kernel(
      body,
      out_type=jax.ShapeDtypeStruct((NC, n_slab, N_PAD, HALF), jnp.float32),
      mesh=mesh,
      scratch_types=[
          pltpu.VMEM((CH_PER_TILE, CHUNK), jnp.int32),
          pltpu.VMEM((CH_PER_TILE, CHUNK), jnp.int32),
          pltpu.VMEM((CHUNK, HALF), jnp.float32),
          pltpu.VMEM((CHUNK, HALF), jnp.float32),
          pltpu.VMEM_SHARED((N_PAD, HALF), jnp.float32),
          pltpu.SemaphoreType.DMA,
          pltpu.SemaphoreType.DMA,
          pltpu.SemaphoreType.DMA,
          pltpu.SemaphoreType.DMA,
      ],
      name="sc_segsum%d%s" % (n_pass, "d" if deg_pass else ""),
  )


_sc_built = {}


def _sc_kernels():
  if not _sc_built:
    mesh = plsc.VectorSubcoreMesh(core_axis_name="c", subcore_axis_name="s",
                                  num_cores=NC, num_subcores=NS)
    _sc_built["segsum"] = _build_sc_segsum(mesh, 2, False)
    _sc_built["segsum_deg"] = _build_sc_segsum(mesh, 2, True)
  return _sc_built["segsum"], _sc_built["segsum_deg"]

_ROWS = 1024
_GRID = N_PAD // _ROWS


def _tc_layer1_body(p_ref, x_ref, wl1_ref, bl1_ref, wr1_ref,
                    wl2_ref, wr2_ref, mlo_ref, mhi_ref, hr2_ref):
  p = p_ref[...]
  agg = jnp.concatenate([p[0, 0] + p[1, 0], p[0, 1] + p[1, 1]], axis=-1)
  dg = (p[0, 2] + p[1, 2])[:, 0:1]
  aggn = agg / jnp.maximum(dg, 1.0)
  h = jnp.maximum(
      jnp.dot(aggn, wl1_ref[...], preferred_element_type=jnp.float32)
      + bl1_ref[...]
      + jnp.dot(x_ref[...], wr1_ref[...], preferred_element_type=jnp.float32),
      0.0)
  m = jnp.dot(h, wl2_ref[...], preferred_element_type=jnp.float32)
  mlo_ref[...] = m[:, :HALF]
  mhi_ref[...] = m[:, HALF:]
  hr2_ref[...] = jnp.dot(h, wr2_ref[...], preferred_element_type=jnp.float32)


def _tc_layer1(P1, x_pad, Wl1T, bl1r, Wr1T, Wl2T, Wr2T):
  return pl.pallas_call(
      _tc_layer1_body,
      grid=(_GRID,),
      in_specs=[
          pl.BlockSpec((NC, 3, _ROWS, HALF), lambda i: (0, 0, i, 0)),
          pl.BlockSpec((_ROWS, D_IN), lambda i: (i, 0)),
          pl.BlockSpec((D_IN, D_H), lambda i: (0, 0)),
          pl.BlockSpec((1, D_H), lambda i: (0, 0)),
          pl.BlockSpec((D_IN, D_H), lambda i: (0, 0)),
          pl.BlockSpec((D_H, D_OUT), lambda i: (0, 0)),
          pl.BlockSpec((D_H, D_OUT), lambda i: (0, 0)),
      ],
      out_specs=[
          pl.BlockSpec((_ROWS, HALF), lambda i: (i, 0)),
          pl.BlockSpec((_ROWS, HALF), lambda i: (i, 0)),
          pl.BlockSpec((_ROWS, D_OUT), lambda i: (i, 0)),
      ],
      out_shape=[
          jax.ShapeDtypeStruct((N_PAD, HALF), jnp.float32),
          jax.ShapeDtypeStruct((N_PAD, HALF), jnp.float32),
          jax.ShapeDtypeStruct((N_PAD, D_OUT), jnp.float32),
      ],
      name="tc_layer1",
  )(P1, x_pad, Wl1T, bl1r, Wr1T, Wl2T, Wr2T)


def _tc_layer2_body(p_ref, deg_ref, hr2_ref, bl2_ref, out_ref):
  p = p_ref[...]
  agg = jnp.concatenate([p[0, 0] + p[1, 0], p[0, 1] + p[1, 1]], axis=-1)
  dg = deg_ref[0] + deg_ref[1]
  out_ref[...] = agg / jnp.maximum(dg, 1.0) + bl2_ref[...] + hr2_ref[...]


def _tc_layer2(P2, deg2, hr2, bl2r):
  return pl.pallas_call(
      _tc_layer2_body,
      grid=(_GRID,),
      in_specs=[
          pl.BlockSpec((NC, 2, _ROWS, HALF), lambda i: (0, 0, i, 0)),
          pl.BlockSpec((NC, _ROWS, 1), lambda i: (0, i, 0)),
          pl.BlockSpec((_ROWS, D_OUT), lambda i: (i, 0)),
          pl.BlockSpec((1, D_OUT), lambda i: (0, 0)),
      ],
      out_specs=pl.BlockSpec((_ROWS, D_OUT), lambda i: (i, 0)),
      out_shape=jax.ShapeDtypeStruct((N_PAD, D_OUT), jnp.float32),
      name="tc_layer2",
  )(P2, deg2, hr2, bl2r)


def kernel(x, edge_index, Wl1, bl1, Wr1, Wl2, bl2, Wr2):
  src = edge_index[0]
  dst = edge_index[1]
  pad = E_PAD - E
  src2 = jnp.concatenate([src, jnp.zeros((pad,), jnp.int32)]).reshape(
      E_PAD // CHUNK, CHUNK)
  dst2 = jnp.concatenate([dst, jnp.full((pad,), TRASH, jnp.int32)]).reshape(
      E_PAD // CHUNK, CHUNK)
  x_lo = x[:, :HALF]
  x_hi = x[:, HALF:]
  x_pad = jnp.concatenate(
      [x, jnp.zeros((N_PAD - N, D_IN), x.dtype)], axis=0)
  zh = jnp.zeros((ROWS_PER_TILE, HALF), jnp.float32)
  ones_h = jnp.ones((CHUNK, HALF), jnp.float32)

  sc_segsum, sc_segsum_deg = _sc_kernels()
  P1 = sc_segsum_deg(x_lo, x_hi, src2, dst2, zh, ones_h)
  deg2 = P1[:, 2, :, 0:1]
  m_lo, m_hi, hr2 = _tc_layer1(P1, x_pad, Wl1.T, bl1.reshape(1, -1),
                               Wr1.T, Wl2.T, Wr2.T)
  P2 = sc_segsum(m_lo, m_hi, src2, dst2, zh, ones_h)
  out_pad = _tc_layer2(P2, deg2, hr2, bl2.reshape(1, -1))
  return out_pad[:N]


# speedup vs baseline: 3.0588x; 1.1607x over previous
import jax
import jax.numpy as jnp
from jax import lax
from jax.experimental import pallas as pl
from jax.experimental.pallas import tpu as pltpu
from jax.experimental.pallas import tpu_sc as plsc

N = 10000
E = 160000
D_IN = 256
D_H = 512
D_OUT = 256

NC = 2
NS = 16
NW = NC * NS
CHUNK = 128
CH0 = 56
CH1 = 24
CH_MAX = max(CH0, CH1)
E_PAD = CHUNK * NS * (CH0 + CH1)
N_PAD = 10240
ROWS_PER_TILE = N_PAD // NS
TRASH = N_PAD - 8
HALF = 128

def _build_sc_segsum(mesh, n_pass, deg_pass):
  n_slab = n_pass + (1 if deg_pass else 0)

  def body(*refs):
    xs = refs[:n_pass]
    (src2, dst2, zrows, ones_h, p_out,
     src_v, dst_v, rows_a, rows_b, acc_sh,
     sem_ga, sem_gb, sem_sa, sem_sb) = refs[n_pass:]
    c = lax.axis_index("c")
    s = lax.axis_index("s")
    nch = jnp.where(c == 0, CH0, CH1)
    @pl.when(c == 0)
    def _():
      pltpu.sync_copy(src2.at[pl.ds(s * CH0, CH0)], src_v.at[pl.ds(0, CH0)])
      pltpu.sync_copy(dst2.at[pl.ds(s * CH0, CH0)], dst_v.at[pl.ds(0, CH0)])

    @pl.when(c == 1)
    def _():
      base = NS * CH0 + s * CH1
      pltpu.sync_copy(src2.at[pl.ds(base, CH1)], src_v.at[pl.ds(0, CH1)])
      pltpu.sync_copy(dst2.at[pl.ds(base, CH1)], dst_v.at[pl.ds(0, CH1)])

    def zero_acc():
      pltpu.sync_copy(zrows, rows_a)
      for k in range(ROWS_PER_TILE // CHUNK):
        pltpu.sync_copy(
            rows_a, acc_sh.at[pl.ds(s * ROWS_PER_TILE + k * CHUNK, CHUNK)])

    def flush(slab):
      rows = pl.ds(s * ROWS_PER_TILE, ROWS_PER_TILE)
      pltpu.sync_copy(acc_sh.at[rows], p_out.at[c, slab, rows])

    for p in range(n_pass):
      xh = xs[p]
      zero_acc()
      plsc.subcore_barrier()

      pltpu.async_copy(xh.at[src_v.at[0]], rows_a, sem_ga)
      pltpu.async_copy(xh.at[src_v.at[1]], rows_b, sem_gb)

      @pl.loop(0, nch, step=2)
      def chunk2(j):
        pltpu.make_async_copy(xh.at[src_v.at[j]], rows_a, sem_ga).wait()
        pltpu.async_copy(rows_a, acc_sh.at[dst_v.at[j]], sem_sa, add=True)
        pltpu.make_async_copy(xh.at[src_v.at[j + 1]], rows_b, sem_gb).wait()
        pltpu.async_copy(rows_b, acc_sh.at[dst_v.at[j + 1]], sem_sb, add=True)
        pltpu.make_async_copy(rows_a, acc_sh.at[dst_v.at[j]], sem_sa).wait()

        @pl.when(j + 2 < nch)
        def _():
          pltpu.async_copy(xh.at[src_v.at[j + 2]], rows_a, sem_ga)

        pltpu.make_async_copy(rows_b, acc_sh.at[dst_v.at[j + 1]], sem_sb).wait()

        @pl.when(j + 3 < nch)
        def _():
          pltpu.async_copy(xh.at[src_v.at[j + 3]], rows_b, sem_gb)

      plsc.subcore_barrier()
      flush(p)

    if deg_pass:
      zero_acc()
      pltpu.sync_copy(ones_h, rows_a)
      pltpu.sync_copy(ones_h, rows_b)
      plsc.subcore_barrier()

      @pl.loop(0, nch, step=2)
      def dchunk(j):
        pltpu.async_copy(rows_a, acc_sh.at[dst_v.at[j]], sem_sa, add=True)
        pltpu.async_copy(rows_b, acc_sh.at[dst_v.at[j + 1]], sem_sb, add=True)
        pltpu.make_async_copy(rows_a, acc_sh.at[dst_v.at[j]], sem_sa).wait()
        pltpu.make_async_copy(rows_b, acc_sh.at[dst_v.at[j + 1]], sem_sb).wait()

      plsc.subcore_barrier()
      flush(n_pass)

  return pl.kernel(
      body,
      out_type=jax.ShapeDtypeStruct((NC, n_slab, N_PAD, HALF), jnp.float32),
      mesh=mesh,
      scratch_types=[
          pltpu.VMEM((CH_MAX, CHUNK), jnp.int32),
          pltpu.VMEM((CH_MAX, CHUNK), jnp.int32),
          pltpu.VMEM((CHUNK, HALF), jnp.float32),
          pltpu.VMEM((CHUNK, HALF), jnp.float32),
          pltpu.VMEM_SHARED((N_PAD, HALF), jnp.float32),
          pltpu.SemaphoreType.DMA,
          pltpu.SemaphoreType.DMA,
          pltpu.SemaphoreType.DMA,
          pltpu.SemaphoreType.DMA,
      ],
      name="sc_segsum%d%s" % (n_pass, "d" if deg_pass else ""),
  )


_sc_built = {}


def _sc_kernels():
  if not _sc_built:
    mesh = plsc.VectorSubcoreMesh(core_axis_name="c", subcore_axis_name="s",
                                  num_cores=NC, num_subcores=NS)
    _sc_built["segsum"] = _build_sc_segsum(mesh, 2, False)
    _sc_built["segsum_deg"] = _build_sc_segsum(mesh, 2, True)
  return _sc_built["segsum"], _sc_built["segsum_deg"]

_ROWS = 1024
_GRID = N_PAD // _ROWS


def _tc_layer1_body(p_ref, x_ref, wl1_ref, bl1_ref, wr1_ref,
                    wl2_ref, wr2_ref, mlo_ref, mhi_ref, hr2_ref):
  p = p_ref[...]
  agg = jnp.concatenate([p[0, 0] + p[1, 0], p[0, 1] + p[1, 1]], axis=-1)
  dg = (p[0, 2] + p[1, 2])[:, 0:1]
  aggn = agg / jnp.maximum(dg, 1.0)
  h = jnp.maximum(
      jnp.dot(aggn, wl1_ref[...], preferred_element_type=jnp.float32)
      + bl1_ref[...]
      + jnp.dot(x_ref[...], wr1_ref[...], preferred_element_type=jnp.float32),
      0.0)
  m = jnp.dot(h, wl2_ref[...], preferred_element_type=jnp.float32)
  mlo_ref[...] = m[:, :HALF]
  mhi_ref[...] = m[:, HALF:]
  hr2_ref[...] = jnp.dot(h, wr2_ref[...], preferred_element_type=jnp.float32)


def _tc_layer1(P1, x_pad, Wl1T, bl1r, Wr1T, Wl2T, Wr2T):
  return pl.pallas_call(
      _tc_layer1_body,
      grid=(_GRID,),
      in_specs=[
          pl.BlockSpec((NC, 3, _ROWS, HALF), lambda i: (0, 0, i, 0)),
          pl.BlockSpec((_ROWS, D_IN), lambda i: (i, 0)),
          pl.BlockSpec((D_IN, D_H), lambda i: (0, 0)),
          pl.BlockSpec((1, D_H), lambda i: (0, 0)),
          pl.BlockSpec((D_IN, D_H), lambda i: (0, 0)),
          pl.BlockSpec((D_H, D_OUT), lambda i: (0, 0)),
          pl.BlockSpec((D_H, D_OUT), lambda i: (0, 0)),
      ],
      out_specs=[
          pl.BlockSpec((_ROWS, HALF), lambda i: (i, 0)),
          pl.BlockSpec((_ROWS, HALF), lambda i: (i, 0)),
          pl.BlockSpec((_ROWS, D_OUT), lambda i: (i, 0)),
      ],
      out_shape=[
          jax.ShapeDtypeStruct((N_PAD, HALF), jnp.float32),
          jax.ShapeDtypeStruct((N_PAD, HALF), jnp.float32),
          jax.ShapeDtypeStruct((N_PAD, D_OUT), jnp.float32),
      ],
      name="tc_layer1",
  )(P1, x_pad, Wl1T, bl1r, Wr1T, Wl2T, Wr2T)


def _tc_layer2_body(p_ref, deg_ref, hr2_ref, bl2_ref, out_ref):
  p = p_ref[...]
  agg = jnp.concatenate([p[0, 0] + p[1, 0], p[0, 1] + p[1, 1]], axis=-1)
  dg = deg_ref[0] + deg_ref[1]
  out_ref[...] = agg / jnp.maximum(dg, 1.0) + bl2_ref[...] + hr2_ref[...]


def _tc_layer2(P2, deg2, hr2, bl2r):
  return pl.pallas_call(
      _tc_layer2_body,
      grid=(_GRID,),
      in_specs=[
          pl.BlockSpec((NC, 2, _ROWS, HALF), lambda i: (0, 0, i, 0)),
          pl.BlockSpec((NC, _ROWS, 1), lambda i: (0, i, 0)),
          pl.BlockSpec((_ROWS, D_OUT), lambda i: (i, 0)),
          pl.BlockSpec((1, D_OUT), lambda i: (0, 0)),
      ],
      out_specs=pl.BlockSpec((_ROWS, D_OUT), lambda i: (i, 0)),
      out_shape=jax.ShapeDtypeStruct((N_PAD, D_OUT), jnp.float32),
      name="tc_layer2",
  )(P2, deg2, hr2, bl2r)


def kernel(x, edge_index, Wl1, bl1, Wr1, Wl2, bl2, Wr2):
  src = edge_index[0]
  dst = edge_index[1]
  pad = E_PAD - E
  src2 = jnp.concatenate([src, jnp.zeros((pad,), jnp.int32)]).reshape(
      E_PAD // CHUNK, CHUNK)
  dst2 = jnp.concatenate([dst, jnp.full((pad,), TRASH, jnp.int32)]).reshape(
      E_PAD // CHUNK, CHUNK)
  x_lo = x[:, :HALF]
  x_hi = x[:, HALF:]
  x_pad = jnp.concatenate(
      [x, jnp.zeros((N_PAD - N, D_IN), x.dtype)], axis=0)
  zh = jnp.zeros((CHUNK, HALF), jnp.float32)
  ones_h = jnp.ones((CHUNK, HALF), jnp.float32)

  sc_segsum, sc_segsum_deg = _sc_kernels()
  P1 = sc_segsum_deg(x_lo, x_hi, src2, dst2, zh, ones_h)
  deg2 = P1[:, 2, :, 0:1]
  m_lo, m_hi, hr2 = _tc_layer1(P1, x_pad, Wl1.T, bl1.reshape(1, -1),
                               Wr1.T, Wl2.T, Wr2.T)
  P2 = sc_segsum(m_lo, m_hi, src2, dst2, zh, ones_h)
  out_pad = _tc_layer2(P2, deg2, hr2, bl2.reshape(1, -1))
  return out_pad[:N]


# speedup vs baseline: 3.2181x; 1.0521x over previous
import jax
import jax.numpy as jnp
from jax import lax
from jax.experimental import pallas as pl
from jax.experimental.pallas import tpu as pltpu
from jax.experimental.pallas import tpu_sc as plsc

N = 10000
E = 160000
D_IN = 256
D_H = 512
D_OUT = 256

NC = 2
NS = 16
NW = NC * NS
CHUNK = 128
CH0 = 64
CH1 = 16
CH_MAX = max(CH0, CH1)
E_PAD = CHUNK * NS * (CH0 + CH1)
N_PAD = 10240
ROWS_PER_TILE = N_PAD // NS
TRASH = N_PAD - 8
HALF = 128

def _build_sc_segsum(mesh, n_pass, deg_pass):
  n_slab = n_pass + (1 if deg_pass else 0)

  def body(*refs):
    xs = refs[:n_pass]
    (src2, dst2, zrows, ones_h, p_out,
     src_v, dst_v, rows_a, rows_b, acc_sh,
     sem_ga, sem_gb, sem_sa, sem_sb) = refs[n_pass:]
    c = lax.axis_index("c")
    s = lax.axis_index("s")
    nch = jnp.where(c == 0, CH0, CH1)
    @pl.when(c == 0)
    def _():
      pltpu.sync_copy(src2.at[pl.ds(s * CH0, CH0)], src_v.at[pl.ds(0, CH0)])
      pltpu.sync_copy(dst2.at[pl.ds(s * CH0, CH0)], dst_v.at[pl.ds(0, CH0)])

    @pl.when(c == 1)
    def _():
      base = NS * CH0 + s * CH1
      pltpu.sync_copy(src2.at[pl.ds(base, CH1)], src_v.at[pl.ds(0, CH1)])
      pltpu.sync_copy(dst2.at[pl.ds(base, CH1)], dst_v.at[pl.ds(0, CH1)])

    def zero_acc():
      pltpu.sync_copy(zrows, rows_a)
      for k in range(ROWS_PER_TILE // CHUNK):
        pltpu.sync_copy(
            rows_a, acc_sh.at[pl.ds(s * ROWS_PER_TILE + k * CHUNK, CHUNK)])

    def flush(slab):
      rows = pl.ds(s * ROWS_PER_TILE, ROWS_PER_TILE)
      pltpu.sync_copy(acc_sh.at[rows], p_out.at[c, slab, rows])

    for p in range(n_pass):
      xh = xs[p]
      zero_acc()
      plsc.subcore_barrier()

      pltpu.async_copy(xh.at[src_v.at[0]], rows_a, sem_ga)
      pltpu.async_copy(xh.at[src_v.at[1]], rows_b, sem_gb)

      @pl.loop(0, nch, step=2)
      def chunk2(j):
        pltpu.make_async_copy(xh.at[src_v.at[j]], rows_a, sem_ga).wait()
        pltpu.async_copy(rows_a, acc_sh.at[dst_v.at[j]], sem_sa, add=True)
        pltpu.make_async_copy(xh.at[src_v.at[j + 1]], rows_b, sem_gb).wait()
        pltpu.async_copy(rows_b, acc_sh.at[dst_v.at[j + 1]], sem_sb, add=True)
        pltpu.make_async_copy(rows_a, acc_sh.at[dst_v.at[j]], sem_sa).wait()

        @pl.when(j + 2 < nch)
        def _():
          pltpu.async_copy(xh.at[src_v.at[j + 2]], rows_a, sem_ga)

        pltpu.make_async_copy(rows_b, acc_sh.at[dst_v.at[j + 1]], sem_sb).wait()

        @pl.when(j + 3 < nch)
        def _():
          pltpu.async_copy(xh.at[src_v.at[j + 3]], rows_b, sem_gb)

      plsc.subcore_barrier()
      flush(p)

    if deg_pass:
      zero_acc()
      pltpu.sync_copy(ones_h, rows_a)
      pltpu.sync_copy(ones_h, rows_b)
      plsc.subcore_barrier()

      @pl.loop(0, nch, step=2)
      def dchunk(j):
        pltpu.async_copy(rows_a, acc_sh.at[dst_v.at[j]], sem_sa, add=True)
        pltpu.async_copy(rows_b, acc_sh.at[dst_v.at[j + 1]], sem_sb, add=True)
        pltpu.make_async_copy(rows_a, acc_sh.at[dst_v.at[j]], sem_sa).wait()
        pltpu.make_async_copy(rows_b, acc_sh.at[dst_v.at[j + 1]], sem_sb).wait()

      plsc.subcore_barrier()
      flush(n_pass)

  return pl.kernel(
      body,
      out_type=jax.ShapeDtypeStruct((NC, n_slab, N_PAD, HALF), jnp.float32),
      mesh=mesh,
      scratch_types=[
          pltpu.VMEM((CH_MAX, CHUNK), jnp.int32),
          pltpu.VMEM((CH_MAX, CHUNK), jnp.int32),
          pltpu.VMEM((CHUNK, HALF), jnp.float32),
          pltpu.VMEM((CHUNK, HALF), jnp.float32),
          pltpu.VMEM_SHARED((N_PAD, HALF), jnp.float32),
          pltpu.SemaphoreType.DMA,
          pltpu.SemaphoreType.DMA,
          pltpu.SemaphoreType.DMA,
          pltpu.SemaphoreType.DMA,
      ],
      name="sc_segsum%d%s" % (n_pass, "d" if deg_pass else ""),
  )


_sc_built = {}


def _sc_kernels():
  if not _sc_built:
    mesh = plsc.VectorSubcoreMesh(core_axis_name="c", subcore_axis_name="s",
                                  num_cores=NC, num_subcores=NS)
    _sc_built["segsum"] = _build_sc_segsum(mesh, 2, False)
    _sc_built["segsum_deg"] = _build_sc_segsum(mesh, 2, True)
  return _sc_built["segsum"], _sc_built["segsum_deg"]

_ROWS = 1024
_GRID = N_PAD // _ROWS


def _tc_layer1_body(p_ref, x_ref, wl1_ref, bl1_ref, wr1_ref,
                    wl2_ref, wr2_ref, mlo_ref, mhi_ref, hr2_ref):
  p = p_ref[...]
  agg = jnp.concatenate([p[0, 0] + p[1, 0], p[0, 1] + p[1, 1]], axis=-1)
  dg = (p[0, 2] + p[1, 2])[:, 0:1]
  aggn = agg / jnp.maximum(dg, 1.0)
  h = jnp.maximum(
      jnp.dot(aggn, wl1_ref[...], preferred_element_type=jnp.float32)
      + bl1_ref[...]
      + jnp.dot(x_ref[...], wr1_ref[...], preferred_element_type=jnp.float32),
      0.0)
  m = jnp.dot(h, wl2_ref[...], preferred_element_type=jnp.float32)
  mlo_ref[...] = m[:, :HALF]
  mhi_ref[...] = m[:, HALF:]
  hr2_ref[...] = jnp.dot(h, wr2_ref[...], preferred_element_type=jnp.float32)


def _tc_layer1(P1, x_pad, Wl1T, bl1r, Wr1T, Wl2T, Wr2T):
  return pl.pallas_call(
      _tc_layer1_body,
      grid=(_GRID,),
      in_specs=[
          pl.BlockSpec((NC, 3, _ROWS, HALF), lambda i: (0, 0, i, 0)),
          pl.BlockSpec((_ROWS, D_IN), lambda i: (i, 0)),
          pl.BlockSpec((D_IN, D_H), lambda i: (0, 0)),
          pl.BlockSpec((1, D_H), lambda i: (0, 0)),
          pl.BlockSpec((D_IN, D_H), lambda i: (0, 0)),
          pl.BlockSpec((D_H, D_OUT), lambda i: (0, 0)),
          pl.BlockSpec((D_H, D_OUT), lambda i: (0, 0)),
      ],
      out_specs=[
          pl.BlockSpec((_ROWS, HALF), lambda i: (i, 0)),
          pl.BlockSpec((_ROWS, HALF), lambda i: (i, 0)),
          pl.BlockSpec((_ROWS, D_OUT), lambda i: (i, 0)),
      ],
      out_shape=[
          jax.ShapeDtypeStruct((N_PAD, HALF), jnp.float32),
          jax.ShapeDtypeStruct((N_PAD, HALF), jnp.float32),
          jax.ShapeDtypeStruct((N_PAD, D_OUT), jnp.float32),
      ],
      name="tc_layer1",
  )(P1, x_pad, Wl1T, bl1r, Wr1T, Wl2T, Wr2T)


def _tc_layer2_body(p_ref, deg_ref, hr2_ref, bl2_ref, out_ref):
  p = p_ref[...]
  agg = jnp.concatenate([p[0, 0] + p[1, 0], p[0, 1] + p[1, 1]], axis=-1)
  dg = deg_ref[0] + deg_ref[1]
  out_ref[...] = agg / jnp.maximum(dg, 1.0) + bl2_ref[...] + hr2_ref[...]


def _tc_layer2(P2, deg2, hr2, bl2r):
  return pl.pallas_call(
      _tc_layer2_body,
      grid=(_GRID,),
      in_specs=[
          pl.BlockSpec((NC, 2, _ROWS, HALF), lambda i: (0, 0, i, 0)),
          pl.BlockSpec((NC, _ROWS, 1), lambda i: (0, i, 0)),
          pl.BlockSpec((_ROWS, D_OUT), lambda i: (i, 0)),
          pl.BlockSpec((1, D_OUT), lambda i: (0, 0)),
      ],
      out_specs=pl.BlockSpec((_ROWS, D_OUT), lambda i: (i, 0)),
      out_shape=jax.ShapeDtypeStruct((N_PAD, D_OUT), jnp.float32),
      name="tc_layer2",
  )(P2, deg2, hr2, bl2r)


def kernel(x, edge_index, Wl1, bl1, Wr1, Wl2, bl2, Wr2):
  src = edge_index[0]
  dst = edge_index[1]
  pad = E_PAD - E
  src2 = jnp.concatenate([src, jnp.zeros((pad,), jnp.int32)]).reshape(
      E_PAD // CHUNK, CHUNK)
  dst2 = jnp.concatenate([dst, jnp.full((pad,), TRASH, jnp.int32)]).reshape(
      E_PAD // CHUNK, CHUNK)
  x_lo = x[:, :HALF]
  x_hi = x[:, HALF:]
  x_pad = jnp.concatenate(
      [x, jnp.zeros((N_PAD - N, D_IN), x.dtype)], axis=0)
  zh = jnp.zeros((CHUNK, HALF), jnp.float32)
  ones_h = jnp.ones((CHUNK, HALF), jnp.float32)

  sc_segsum, sc_segsum_deg = _sc_kernels()
  P1 = sc_segsum_deg(x_lo, x_hi, src2, dst2, zh, ones_h)
  deg2 = P1[:, 2, :, 0:1]
  m_lo, m_hi, hr2 = _tc_layer1(P1, x_pad, Wl1.T, bl1.reshape(1, -1),
                               Wr1.T, Wl2.T, Wr2.T)
  P2 = sc_segsum(m_lo, m_hi, src2, dst2, zh, ones_h)
  out_pad = _tc_layer2(P2, deg2, hr2, bl2.reshape(1, -1))
  return out_pad[:N]
